# baseline (speedup 1.0000x reference)
import functools

import jax
import jax.numpy as jnp
from jax import lax
from jax.experimental import pallas as pl
from jax.experimental.pallas import tpu as pltpu
from jax.experimental.pallas import tpu_sc as plsc

_B, _L, _D = 4096, 50, 64
_N = _B * _L
_NW = 32
_BPW = _N // _NW
_S = 128
_NSTREAM = _BPW // _S

_mesh = plsc.VectorSubcoreMesh(core_axis_name="c", subcore_axis_name="s")


@functools.partial(
    pl.kernel,
    mesh=_mesh,
    out_type=jax.ShapeDtypeStruct((_N, _D), jnp.float32),
    scratch_types=[
        pltpu.VMEM((_NSTREAM, _S), jnp.int32),
        pltpu.VMEM((_S, _D), jnp.float32),
        pltpu.SemaphoreType.DMA,
    ],
    compiler_params=pltpu.CompilerParams(use_tc_tiling_on_sc=False),
)
def _gather(table_hbm, idx_hbm, out_hbm, idx_v, rows_v, sem):
    wid = lax.axis_index("s") * 2 + lax.axis_index("c")
    base = wid * _BPW
    pltpu.sync_copy(idx_hbm.at[wid], idx_v)

    def step(j, carry):
        pltpu.async_copy(table_hbm.at[idx_v.at[j]], rows_v, sem).wait()
        pltpu.sync_copy(rows_v, out_hbm.at[pl.ds(base + j * _S, _S)])
        return carry

    lax.fori_loop(0, _NSTREAM, step, 0)


def kernel(source, table):
    idx = source.reshape(_NW, _NSTREAM, _S)
    out = _gather(table, idx)
    return out.reshape(_B, _L, _D)


# speedup vs baseline: 1.0469x; 1.0469x over previous
import functools

import jax
import jax.numpy as jnp
from jax import lax
from jax.experimental import pallas as pl
from jax.experimental.pallas import tpu as pltpu
from jax.experimental.pallas import tpu_sc as plsc

_B, _L, _D = 4096, 50, 64
_N = _B * _L
_NW = 32
_BPW = _N // _NW
_S = 128
_NSTREAM = _BPW // _S

_K = 5
_G = _NSTREAM // _K
_GR = _K * _S

_mesh = plsc.VectorSubcoreMesh(core_axis_name="c", subcore_axis_name="s")


@functools.partial(
    pl.kernel,
    mesh=_mesh,
    out_type=jax.ShapeDtypeStruct((_N, _D), jnp.float32),
    scratch_types=[
        pltpu.VMEM((_G, _GR), jnp.int32),
        pltpu.VMEM((_GR, _D), jnp.float32),
        pltpu.VMEM((_GR, _D), jnp.float32),
        pltpu.SemaphoreType.DMA,
        pltpu.SemaphoreType.DMA,
    ],
    compiler_params=pltpu.CompilerParams(use_tc_tiling_on_sc=False),
)
def _gather(table_hbm, idx_hbm, out_hbm, idx_v, rows0, rows1, sem0, sem1):
    wid = lax.axis_index("s") * 2 + lax.axis_index("c")
    base = wid * _BPW
    pltpu.sync_copy(idx_hbm.at[wid], idx_v)

    bufs = (rows0, rows1)
    sems = (sem0, sem1)
    copies = [None, None]
    for g in range(_G):
        b = g % 2
        copies[b] = pltpu.async_copy(table_hbm.at[idx_v.at[g]], bufs[b], sems[b])
        if g >= 1:
            pb = (g - 1) % 2
            copies[pb].wait()
            pltpu.sync_copy(bufs[pb], out_hbm.at[pl.ds(base + (g - 1) * _GR, _GR)])
    lb = (_G - 1) % 2
    copies[lb].wait()
    pltpu.sync_copy(bufs[lb], out_hbm.at[pl.ds(base + (_G - 1) * _GR, _GR)])


def kernel(source, table):
    idx = source.reshape(_NW, _G, _GR)
    out = _gather(table, idx)
    return out.reshape(_B, _L, _D)


# speedup vs baseline: 1.0516x; 1.0045x over previous
import functools

import jax
import jax.numpy as jnp
from jax import lax
from jax.experimental import pallas as pl
from jax.experimental.pallas import tpu as pltpu
from jax.experimental.pallas import tpu_sc as plsc

_B, _L, _D = 4096, 50, 64
_DP = 128
_N = _B * _L
_NW = 32
_BPW = _N // _NW
_GR = 320
_G = _BPW // _GR

_mesh = plsc.VectorSubcoreMesh(core_axis_name="c", subcore_axis_name="s")


@functools.partial(
    pl.kernel,
    mesh=_mesh,
    out_type=jax.ShapeDtypeStruct((_N, _DP), jnp.float32),
    scratch_types=[
        pltpu.VMEM((_BPW,), jnp.int32),
        pltpu.VMEM((_GR, _DP), jnp.float32),
        pltpu.VMEM((_GR, _DP), jnp.float32),
        pltpu.SemaphoreType.DMA,
        pltpu.SemaphoreType.DMA,
    ],
    compiler_params=pltpu.CompilerParams(use_tc_tiling_on_sc=True),
)
def _gather(table_hbm, idx_hbm, out_hbm, idx_v, rows0, rows1, sem0, sem1):
    wid = lax.axis_index("s") * 2 + lax.axis_index("c")
    base = wid * _BPW
    pltpu.sync_copy(idx_hbm.at[wid], idx_v)

    bufs = (rows0, rows1)
    sems = (sem0, sem1)
    copies = [None, None]
    for g in range(_G):
        b = g % 2
        copies[b] = pltpu.async_copy(table_hbm.at[idx_v.at[pl.ds(g * _GR, _GR)]], bufs[b], sems[b])
        if g >= 1:
            pb = (g - 1) % 2
            copies[pb].wait()
            pltpu.sync_copy(bufs[pb], out_hbm.at[pl.ds(base + (g - 1) * _GR, _GR)])
    lb = (_G - 1) % 2
    copies[lb].wait()
    pltpu.sync_copy(bufs[lb], out_hbm.at[pl.ds(base + (_G - 1) * _GR, _GR)])


def kernel(source, table):
    tpad = jnp.pad(table, ((0, 0), (0, _DP - _D)))
    idx = source.reshape(_NW, _BPW)
    out = _gather(tpad, idx)
    return lax.slice(out, (0, 0), (_N, _D)).reshape(_B, _L, _D)
